# baseline (speedup 1.0000x reference)
import functools
import jax
import jax.numpy as jnp
from jax import lax
from jax.experimental import pallas as pl
from jax.experimental.pallas import tpu as pltpu
from jax.experimental.pallas import tpu_sc as plsc

B, S, C = 4, 8192, 2048
K = 64
NL = 16
HS = S // 2
NW = 32
NJ = (B * (C // NL)) // NW

def _flip(v):
    k = plsc.bitcast(v, jnp.int32)
    return k ^ ((k >> 31) & 2147483647)


def _unflip(k):
    return plsc.bitcast(k ^ ((k >> 31) & 2147483647), jnp.float32)


def _oem_merge_list(a, b, desc):
    if len(a) == 1:
        mx, mn = jnp.maximum(a[0], b[0]), jnp.minimum(a[0], b[0])
        return [mx, mn] if desc else [mn, mx]
    e = _oem_merge_list(a[0::2], b[0::2], desc)
    o = _oem_merge_list(a[1::2], b[1::2], desc)
    z = [e[0]]
    for i in range(len(o) - 1):
        hi, lo = jnp.maximum(o[i], e[i + 1]), jnp.minimum(o[i], e[i + 1])
        z += [hi, lo] if desc else [lo, hi]
    z.append(o[-1])
    return z


def _oems_sort_list(lst, desc):
    if len(lst) == 1:
        return lst
    h = len(lst) // 2
    return _oem_merge_list(_oems_sort_list(lst[:h], desc),
                           _oems_sort_list(lst[h:], desc), desc)


def _bitonic_merge_list(p, desc):
    n = len(p)
    if n == 1:
        return p
    h = n // 2
    hi = [jnp.maximum(p[i], p[i + h]) for i in range(h)]
    lo = [jnp.minimum(p[i], p[i + h]) for i in range(h)]
    if desc:
        return _bitonic_merge_list(hi, True) + _bitonic_merge_list(lo, True)
    return _bitonic_merge_list(lo, False) + _bitonic_merge_list(hi, False)


def _select_top64(slab, hist, lanes):
    zero = jnp.zeros((NL,), jnp.int32)
    one = jnp.ones((NL,), jnp.int32)
    prefix, above, done = zero, zero, zero

    for lvl in range(4):
        shift = 24 - 8 * lvl
        pmask = -(1 << (shift + 8)) if lvl > 0 else 0

        def do_level(_, prefix=prefix, above=above, done=done,
                     shift=shift, pmask=pmask, lvl=lvl):
            def zero_hist(i, c):
                hist[i] = zero
                return c
            lax.fori_loop(0, 256, zero_hist, 0)

            def scan_body(s, c):
                key = _flip(slab[s])
                if lvl == 0:
                    match = one
                    digit = ((key >> 24) & 255) ^ 128
                else:
                    match = jnp.where((key & pmask) == (prefix & pmask),
                                      1, 0)
                    digit = (key >> shift) & 255
                plsc.addupdate_scatter(hist, [digit, lanes], match)
                return c
            lax.fori_loop(0, HS, scan_body, 0)

            rank = 64 - above

            def find_body(i, c):
                cum, bucket, abv, found = c
                d = 255 - i
                h = hist[d]
                ncum = cum + h
                newly = jnp.where((found == 0) & (ncum >= rank), 1, 0)
                bucket = jnp.where(newly == 1, d, bucket)
                abv = jnp.where(newly == 1, cum, abv)
                return ncum, bucket, abv, found | newly

            _, bucket, babv, _ = lax.fori_loop(
                0, 256, find_body, (zero, zero, zero, zero))
            bcnt = plsc.load_gather(hist, [bucket, lanes])
            keybits = (bucket ^ 128) if lvl == 0 else bucket
            dmask = done == 1
            nprefix = jnp.where(dmask, prefix, prefix | (keybits << shift))
            nabove = jnp.where(dmask, above, above + babv)
            ndone = done | jnp.where(nabove + bcnt == 64, 1, 0)
            return nprefix, nabove, ndone

        def skip(_, prefix=prefix, above=above, done=done):
            return prefix, above, done

        all_done = jnp.min(done) > 0
        prefix, above, done = lax.cond(all_done, skip, do_level, 0)

    return prefix, above


def _sc_kernel(x_hbm, out_hbm, slab, hist, obuf, sem):
    wid = lax.axis_index("s") * 2 + lax.axis_index("c")
    lanes = lax.iota(jnp.int32, 16)

    def job_body(j, carry):
        job = wid * NJ + j
        b = job // (C // NL)
        c0 = (job % (C // NL)) * NL
        halves = []
        for hidx in range(2):
            pltpu.sync_copy(
                x_hbm.at[b, pl.ds(hidx * HS, HS), pl.ds(c0, NL)], slab)
            thr, _n_above = _select_top64(slab, hist, lanes)
            thr_val = _unflip(thr)

            def fill_body(i, c, thr_val=thr_val):
                obuf[i] = thr_val
                return c
            lax.fori_loop(0, K, fill_body, 0)

            def collect_body(s, pos, thr=thr):
                v = slab[s]
                m = _flip(v) > thr
                plsc.store_scatter(obuf, [pos, lanes], v, mask=m)
                return pos + jnp.where(m, 1, 0)
            lax.fori_loop(0, HS, collect_body, jnp.zeros((NL,), jnp.int32))

            pieces = [obuf[i] for i in range(K)]
            halves.append(_oems_sort_list(pieces, hidx == 0))
        m = [jnp.maximum(halves[0][i], halves[1][i]) for i in range(K)]
        top = _bitonic_merge_list(m, True)
        for i in range(K):
            obuf[i] = top[i]
        pltpu.sync_copy(obuf, out_hbm.at[b, :, pl.ds(c0, NL)])
        return carry

    lax.fori_loop(0, NJ, job_body, 0)


def kernel(inputs):
    f = pl.kernel(
        _sc_kernel,
        mesh=plsc.VectorSubcoreMesh(core_axis_name="c", subcore_axis_name="s"),
        compiler_params=pltpu.CompilerParams(use_tc_tiling_on_sc=False, needs_layout_passes=False),
        out_type=jax.ShapeDtypeStruct((B, K, C), jnp.float32),
        scratch_types=[
            pltpu.VMEM((HS, NL), jnp.float32),
            pltpu.VMEM((256, NL), jnp.int32),
            pltpu.VMEM((K, NL), jnp.float32),
            pltpu.SemaphoreType.DMA,
        ],
    )
    return f(inputs)


# speedup vs baseline: 19.0109x; 19.0109x over previous
import jax
import jax.numpy as jnp
from jax.experimental import pallas as pl

S = 8192
K = 64
BC = 128
G0 = S // K


def _oem_merge(a, b, desc):
    if len(a) == 1:
        mx, mn = jnp.maximum(a[0], b[0]), jnp.minimum(a[0], b[0])
        return [mx, mn] if desc else [mn, mx]
    e = _oem_merge(a[0::2], b[0::2], desc)
    o = _oem_merge(a[1::2], b[1::2], desc)
    z = [e[0]]
    for i in range(len(o) - 1):
        hi = jnp.maximum(o[i], e[i + 1])
        lo = jnp.minimum(o[i], e[i + 1])
        z += [hi, lo] if desc else [lo, hi]
    z.append(o[-1])
    return z


def _oems_sort(lst, desc):
    if len(lst) == 1:
        return lst
    h = len(lst) // 2
    return _oem_merge(_oems_sort(lst[:h], desc), _oems_sort(lst[h:], desc), desc)


def _bitonic_merge(p, desc):
    n = len(p)
    if n == 1:
        return p
    h = n // 2
    hi = [jnp.maximum(p[i], p[i + h]) for i in range(h)]
    lo = [jnp.minimum(p[i], p[i + h]) for i in range(h)]
    if desc:
        return _bitonic_merge(hi, True) + _bitonic_merge(lo, True)
    return _bitonic_merge(lo, False) + _bitonic_merge(hi, False)


def _topk_kernel(x_ref, o_ref):
    x = x_ref[0]
    left = [x[i * G0:i * G0 + G0 // 2] for i in range(K)]
    right = [x[i * G0 + G0 // 2:(i + 1) * G0] for i in range(K)]
    cur_d = _oems_sort(left, True)
    cur_a = _oems_sort(right, False)
    g = G0 // 2
    while True:
        m = [jnp.maximum(cur_d[i], cur_a[i]) for i in range(K)]
        if g == 1:
            o_ref[0] = jnp.concatenate(_bitonic_merge(m, True), axis=0)
            return
        h = g // 2
        cur_d = _bitonic_merge([p[:h] for p in m], True)
        cur_a = _bitonic_merge([p[h:] for p in m], False)
        g = h


def kernel(inputs):
    B, s, C = inputs.shape
    assert s == S and C % BC == 0
    grid = (B, C // BC)
    return pl.pallas_call(
        _topk_kernel,
        grid=grid,
        in_specs=[pl.BlockSpec((1, S, BC), lambda b, c: (b, 0, c))],
        out_specs=pl.BlockSpec((1, K, BC), lambda b, c: (b, 0, c)),
        out_shape=jax.ShapeDtypeStruct((B, K, C), jnp.float32),
    )(inputs)


# speedup vs baseline: 19.0629x; 1.0027x over previous
import jax
import jax.numpy as jnp
from jax.experimental import pallas as pl

S = 8192
K = 64
BC = 256
G0 = S // K


def _oem_merge(a, b, desc):
    if len(a) == 1:
        mx, mn = jnp.maximum(a[0], b[0]), jnp.minimum(a[0], b[0])
        return [mx, mn] if desc else [mn, mx]
    e = _oem_merge(a[0::2], b[0::2], desc)
    o = _oem_merge(a[1::2], b[1::2], desc)
    z = [e[0]]
    for i in range(len(o) - 1):
        hi = jnp.maximum(o[i], e[i + 1])
        lo = jnp.minimum(o[i], e[i + 1])
        z += [hi, lo] if desc else [lo, hi]
    z.append(o[-1])
    return z


def _oems_sort(lst, desc):
    if len(lst) == 1:
        return lst
    h = len(lst) // 2
    return _oem_merge(_oems_sort(lst[:h], desc), _oems_sort(lst[h:], desc), desc)


def _bitonic_merge(p, desc):
    n = len(p)
    if n == 1:
        return p
    h = n // 2
    hi = [jnp.maximum(p[i], p[i + h]) for i in range(h)]
    lo = [jnp.minimum(p[i], p[i + h]) for i in range(h)]
    if desc:
        return _bitonic_merge(hi, True) + _bitonic_merge(lo, True)
    return _bitonic_merge(lo, False) + _bitonic_merge(hi, False)


def _topk_kernel(x_ref, o_ref):
    x = x_ref[0]
    left = [x[i * G0:i * G0 + G0 // 2] for i in range(K)]
    right = [x[i * G0 + G0 // 2:(i + 1) * G0] for i in range(K)]
    cur_d = _oems_sort(left, True)
    cur_a = _oems_sort(right, False)
    g = G0 // 2
    while True:
        m = [jnp.maximum(cur_d[i], cur_a[i]) for i in range(K)]
        if g == 1:
            o_ref[0] = jnp.concatenate(_bitonic_merge(m, True), axis=0)
            return
        h = g // 2
        cur_d = _bitonic_merge([p[:h] for p in m], True)
        cur_a = _bitonic_merge([p[h:] for p in m], False)
        g = h


def kernel(inputs):
    B, s, C = inputs.shape
    assert s == S and C % BC == 0
    grid = (B, C // BC)
    return pl.pallas_call(
        _topk_kernel,
        grid=grid,
        in_specs=[pl.BlockSpec((1, S, BC), lambda b, c: (b, 0, c))],
        out_specs=pl.BlockSpec((1, K, BC), lambda b, c: (b, 0, c)),
        out_shape=jax.ShapeDtypeStruct((B, K, C), jnp.float32),
    )(inputs)
